# baseline (speedup 1.0000x reference)
import functools

import jax
import jax.numpy as jnp
from jax import lax
from jax.experimental import pallas as pl
from jax.experimental.pallas import tpu as pltpu
from jax.experimental.pallas import tpu_sc as plsc

NC = 2
NS = 16
NW = NC * NS
CH = 128


def _make_sc_embed(batch, fields, dim):
    bpw = batch // NW
    cpf = bpw // CH
    n_rows = fields * cpf
    mesh = plsc.VectorSubcoreMesh(core_axis_name="c", subcore_axis_name="s")

    @functools.partial(
        pl.kernel,
        out_type=jax.ShapeDtypeStruct((fields, dim, batch), jnp.float32),
        mesh=mesh,
        scratch_types=[
            pltpu.VMEM((CH,), jnp.int32),
            pltpu.VMEM((CH,), jnp.int32),
            pltpu.VMEM((CH,), jnp.int32),
            pltpu.VMEM((CH,), jnp.int32),
            pltpu.VMEM((CH,), jnp.int32),
            pltpu.VMEM((CH,), jnp.int32),
            pltpu.VMEM((CH, 128), jnp.float32),
            pltpu.VMEM((CH, 128), jnp.float32),
            pltpu.VMEM((dim, bpw), jnp.float32),
            pltpu.SemaphoreType.DMA,
            pltpu.SemaphoreType.DMA,
            pltpu.SemaphoreType.DMA,
        ],
        compiler_params=pltpu.CompilerParams(
            use_tc_tiling_on_sc=True, needs_layout_passes=False
        ),
    )
    def body(xw_hbm, e4_hbm, out_hbm,
             raw0, raw1, gr0, gr1, of0, of1, g0, g1, tbuf,
             gs0, gs1, osem):
        wid = lax.axis_index("s") * NC + lax.axis_index("c")
        raw = (raw0, raw1)
        grow = (gr0, gr1)
        goff = (of0, of1)
        gbuf = (g0, g1)
        gsem = (gs0, gs1)
        iota = lax.iota(jnp.int32, 16)

        def fire(row, p):
            pltpu.sync_copy(xw_hbm.at[wid, row], raw[p])
            for g in range(8):
                sl = pl.ds(g * 16, 16)
                xv = raw[p][sl]
                grow[p][sl] = xv >> 2
                goff[p][sl] = (xv & 3) << 5
            pltpu.async_copy(e4_hbm.at[grow[p]], gbuf[p], gsem[p])

        def drain(p):
            pltpu.make_async_copy(e4_hbm.at[pl.ds(0, CH)], gbuf[p], gsem[p]).wait()

        def extract(c, p):
            for g in range(8):
                off = goff[p][pl.ds(g * 16, 16)]
                rows = iota + (g * 16)
                for d in range(dim):
                    vals = plsc.load_gather(gbuf[p], [rows, off + d])
                    tbuf[d, pl.ds(c * CH + g * 16, 16)] = vals

        fire(0, 0)

        def field_step(f, carry):
            for c in range(cpf):
                nxt = f * cpf + c + 1

                @pl.when(nxt < n_rows)
                def _():
                    fire(nxt, (c + 1) % 2)

                drain(c % 2)

                if c == 0:
                    @pl.when(f >= 1)
                    def _():
                        pltpu.make_async_copy(
                            tbuf, out_hbm.at[0, :, pl.ds(0, bpw)], osem
                        ).wait()

                extract(c, c % 2)
            pltpu.async_copy(
                tbuf, out_hbm.at[f, :, pl.ds(wid * bpw, bpw)], osem
            )
            return carry

        lax.fori_loop(0, fields, field_step, 0)
        pltpu.make_async_copy(tbuf, out_hbm.at[0, :, pl.ds(0, bpw)], osem).wait()

    return body


def kernel(x, E):
    b, f = x.shape
    v, d = E.shape
    bpw = b // NW
    cpf = bpw // CH
    xw = (
        x.astype(jnp.int32)
        .T.reshape(f, NW, cpf, CH)
        .swapaxes(0, 1)
        .reshape(NW, f * cpf, CH)
    )
    e4 = E.reshape(v // 4, 4 * d)
    out = _make_sc_embed(b, f, d)(xw, e4)
    return out.transpose(2, 0, 1)


# speedup vs baseline: 1.1281x; 1.1281x over previous
import functools

import jax
import jax.numpy as jnp
from jax import lax
from jax.experimental import pallas as pl
from jax.experimental.pallas import tpu as pltpu
from jax.experimental.pallas import tpu_sc as plsc

NC = 2
NS = 16
NW = NC * NS
CHUNK = 128
K = 13


def _make_sc_gather(n_total, dim):
    per_w = n_total // NW
    n_chunks = per_w // CHUNK
    n_groups = n_chunks // K
    rows_per_group = K * CHUNK
    mesh = plsc.VectorSubcoreMesh(core_axis_name="c", subcore_axis_name="s")

    @functools.partial(
        pl.kernel,
        out_type=jax.ShapeDtypeStruct((n_total, dim), jnp.float32),
        mesh=mesh,
        scratch_types=[
            pltpu.VMEM((n_chunks, CHUNK), jnp.int32),
            pltpu.VMEM((rows_per_group, dim), jnp.float32),
            pltpu.VMEM((rows_per_group, dim), jnp.float32),
            pltpu.SemaphoreType.DMA,
            pltpu.SemaphoreType.DMA,
            pltpu.SemaphoreType.DMA,
            pltpu.SemaphoreType.DMA,
        ],
        compiler_params=pltpu.CompilerParams(use_tc_tiling_on_sc=False),
    )
    def body(idx_hbm, tab_hbm, out_hbm, idx_v, rows0, rows1, g0, g1, o0, o1):
        wid = lax.axis_index("s") * NC + lax.axis_index("c")
        base = wid * per_w
        pltpu.sync_copy(idx_hbm.at[wid], idx_v)

        rows = (rows0, rows1)
        gsem = (g0, g1)
        osem = (o0, o1)

        def fire_gather(g, b):
            for j in range(K):
                pltpu.async_copy(
                    tab_hbm.at[idx_v.at[g * K + j]],
                    rows[b].at[pl.ds(j * CHUNK, CHUNK)],
                    gsem[b],
                )

        def drain_gather(b):
            pltpu.make_async_copy(tab_hbm.at[pl.ds(0, rows_per_group)],
                                  rows[b], gsem[b]).wait()

        def fire_out(g, b):
            pltpu.async_copy(
                rows[b],
                out_hbm.at[pl.ds(base + g * rows_per_group, rows_per_group)],
                osem[b],
            )

        def wait_out(b):
            pltpu.make_async_copy(rows[b],
                                  out_hbm.at[pl.ds(base, rows_per_group)],
                                  osem[b]).wait()

        fire_gather(0, 0)

        def step(g, carry):
            b = g % 2

            def one(bb):
                drain_gather(bb)
                fire_out(g, bb)

                @pl.when(g + 1 < n_groups)
                def _():
                    @pl.when(g >= 1)
                    def _():
                        wait_out(1 - bb)
                    fire_gather(g + 1, 1 - bb)

            @pl.when(b == 0)
            def _():
                one(0)

            @pl.when(b == 1)
            def _():
                one(1)

            return carry

        lax.fori_loop(0, n_groups, step, 0)
        wait_out(0)
        wait_out(1)

    return body


def kernel(x, E):
    b, f = x.shape
    v, d = E.shape
    n_total = b * f
    xf = x.astype(jnp.int32).reshape(NW, n_total // (NW * CHUNK), CHUNK)
    out = _make_sc_gather(n_total, d)(xf, E)
    return out.reshape(b, f, d)
